# baseline (speedup 1.0000x reference)
import functools

import jax
import jax.numpy as jnp
from jax import lax
from jax.experimental import pallas as pl
from jax.experimental.pallas import tpu as pltpu
from jax.experimental.pallas import tpu_sc as plsc

N, D, E, F, S, K = 2048, 1024, 16, 512, 2, 2
FS = F * S
T = 256
NT = (N * K) // T + E
P = NT * T
TB = 256
NC, NS = 2, 16
NW = NC * NS
CH = 64

_NT_DIMS = (((1,), (1,)), ((), ()))


def _silu(v):
    return v * (1.0 / (1.0 + jnp.exp(-v)))


def _route_kernel(x_ref, wg_ref, pos_ref, w_ref, tile_e_ref, tile_v_ref):
    x = x_ref[...]
    wg = wg_ref[...]
    logits = lax.dot_general(x, wg, _NT_DIMS,
                             preferred_element_type=jnp.float32)
    m = jnp.max(logits, axis=1, keepdims=True)
    ex = jnp.exp(logits - m)
    probs = ex / jnp.sum(ex, axis=1, keepdims=True)

    eidx = lax.broadcasted_iota(jnp.int32, (N, E), 1)
    p1 = jnp.max(probs, axis=1, keepdims=True)
    e1 = jnp.min(jnp.where(probs == p1, eidx, E), axis=1, keepdims=True)
    oh1 = eidx == e1
    probs_m = jnp.where(oh1, -1.0, probs)
    p2 = jnp.max(probs_m, axis=1, keepdims=True)
    e2 = jnp.min(jnp.where(probs_m == p2, eidx, E), axis=1, keepdims=True)
    oh2 = eidx == e2
    wsum = p1 + p2
    w_ref[...] = jnp.concatenate([p1 / wsum, p2 / wsum], axis=1)

    c = (oh1 | oh2).astype(jnp.bfloat16)
    tril = (lax.broadcasted_iota(jnp.int32, (N, N), 0)
            >= lax.broadcasted_iota(jnp.int32, (N, N), 1)).astype(jnp.bfloat16)
    s_inc = lax.dot_general(tril, c, (((1,), (0,)), ((), ())),
                            preferred_element_type=jnp.float32)
    counts = s_inc[N - 1:N, :]
    tiles_pe = jnp.ceil(counts * (1.0 / T))
    excl = (lax.broadcasted_iota(jnp.int32, (E, E), 0)
            < lax.broadcasted_iota(jnp.int32, (E, E), 1)).astype(jnp.float32)
    tile_start = lax.dot_general(tiles_pe, excl, (((1,), (0,)), ((), ())),
                                 preferred_element_type=jnp.float32)
    total_tiles = jnp.sum(tiles_pe, axis=1, keepdims=True)
    slot_off = tile_start * float(T)

    rank1 = jnp.sum(jnp.where(oh1, s_inc, 0.0), axis=1, keepdims=True) - 1.0
    rank2 = jnp.sum(jnp.where(oh2, s_inc, 0.0), axis=1, keepdims=True) - 1.0
    off_b = jnp.broadcast_to(slot_off, (N, E))
    off1 = jnp.sum(jnp.where(oh1, off_b, 0.0), axis=1, keepdims=True)
    off2 = jnp.sum(jnp.where(oh2, off_b, 0.0), axis=1, keepdims=True)
    pos = jnp.concatenate([jnp.transpose(off1 + rank1),
                           jnp.transpose(off2 + rank2)], axis=0)
    pos_ref[...] = pos.astype(jnp.int32)

    ends = jnp.transpose(tile_start + tiles_pe)
    gi = lax.broadcasted_iota(jnp.int32, (E, NT), 1).astype(jnp.float32)
    g2 = jnp.minimum(gi, total_tiles[0, 0] - 1.0)
    tile_e = jnp.sum((ends <= g2).astype(jnp.float32), axis=0, keepdims=True)
    tile_e_ref[...] = tile_e.astype(jnp.int32)
    tile_v = (lax.broadcasted_iota(jnp.int32, (1, NT), 1).astype(jnp.float32)
              < total_tiles[0, 0])
    tile_v_ref[...] = tile_v.astype(jnp.int32)


_route = pl.pallas_call(
    _route_kernel,
    out_shape=[
        jax.ShapeDtypeStruct((K, N), jnp.int32),
        jax.ShapeDtypeStruct((N, K), jnp.float32),
        jax.ShapeDtypeStruct((1, NT), jnp.int32),
        jax.ShapeDtypeStruct((1, NT), jnp.int32),
    ],
)


def _sc_scatter_x(x, pos_flat):
    n_per_w = N // NW
    nch = n_per_w // CH
    mesh = plsc.VectorSubcoreMesh(core_axis_name="c", subcore_axis_name="s")

    @functools.partial(
        pl.kernel,
        mesh=mesh,
        out_type=jax.ShapeDtypeStruct((P, D), jnp.float32),
        scratch_types=[
            pltpu.VMEM((CH,), jnp.int32),
            pltpu.VMEM((CH,), jnp.int32),
            pltpu.VMEM((CH, D), jnp.float32),
            pltpu.SemaphoreType.DMA,
        ],
    )
    def scatter(x_hbm, pos_hbm, out_hbm, i0_v, i1_v, rows_v, sem):
        wid = lax.axis_index("s") * NC + lax.axis_index("c")
        base = wid * n_per_w
        for ci in range(nch):
            off = base + ci * CH
            pltpu.sync_copy(pos_hbm.at[pl.ds(off, CH)], i0_v)
            pltpu.sync_copy(pos_hbm.at[pl.ds(N + off, CH)], i1_v)
            pltpu.sync_copy(x_hbm.at[pl.ds(off, CH)], rows_v)
            pltpu.async_copy(rows_v, out_hbm.at[i0_v], sem).wait()
            pltpu.async_copy(rows_v, out_hbm.at[i1_v], sem).wait()

    return scatter(x, pos_flat)


def _sc_gather_rows(table, idx):
    b = idx.shape[0]
    b_per_w = b // NW
    ch = min(b_per_w, 64)
    nch = b_per_w // ch
    mesh = plsc.VectorSubcoreMesh(core_axis_name="c", subcore_axis_name="s")

    @functools.partial(
        pl.kernel,
        mesh=mesh,
        out_type=jax.ShapeDtypeStruct((b, D), jnp.float32),
        scratch_types=[
            pltpu.VMEM((ch,), jnp.int32),
            pltpu.VMEM((ch, D), jnp.float32),
            pltpu.SemaphoreType.DMA,
        ],
    )
    def gather(table_hbm, idx_hbm, out_hbm, idx_v, rows_v, sem):
        wid = lax.axis_index("s") * NC + lax.axis_index("c")
        base = wid * b_per_w
        for ci in range(nch):
            off = base + ci * ch
            pltpu.sync_copy(idx_hbm.at[pl.ds(off, ch)], idx_v)
            pltpu.async_copy(table_hbm.at[idx_v], rows_v, sem).wait()
            pltpu.sync_copy(rows_v, out_hbm.at[pl.ds(off, ch)])

    return gather(table, idx)


FB = 128
NFB = F // FB


def _expert_kernel(tile_e_s, tile_v_s, xg_ref, w1_ref, w2_ref, y_ref):
    g = pl.program_id(0)
    j = pl.program_id(1)

    @pl.when(tile_v_s[g] != 0)
    def _():
        xgb = xg_ref[...].astype(jnp.bfloat16)
        h = _silu(lax.dot_general(xgb, w1_ref[0].astype(jnp.bfloat16),
                                  _NT_DIMS, preferred_element_type=jnp.float32))
        yb = lax.dot_general(h.astype(jnp.bfloat16),
                             w2_ref[0].astype(jnp.bfloat16), _NT_DIMS,
                             preferred_element_type=jnp.float32)

        @pl.when(j == 0)
        def _():
            y_ref[...] = yb

        @pl.when(j != 0)
        def _():
            y_ref[...] = y_ref[...] + yb


_experts = pl.pallas_call(
    _expert_kernel,
    grid_spec=pltpu.PrefetchScalarGridSpec(
        num_scalar_prefetch=2,
        grid=(NT, NFB),
        in_specs=[
            pl.BlockSpec((T, D), lambda g, j, te, tv: (g, 0)),
            pl.BlockSpec((1, FB, D), lambda g, j, te, tv: (te[g], j, 0)),
            pl.BlockSpec((1, D, FB), lambda g, j, te, tv: (te[g], 0, j)),
        ],
        out_specs=pl.BlockSpec((T, D), lambda g, j, te, tv: (g, 0)),
    ),
    out_shape=jax.ShapeDtypeStruct((P, D), jnp.float32),
)


def _shared_kernel(x_ref, ws1_ref, ws2_ref, sh_ref):
    xb = x_ref[...].astype(jnp.bfloat16)
    h = _silu(lax.dot_general(xb, ws1_ref[...].astype(jnp.bfloat16), _NT_DIMS,
                              preferred_element_type=jnp.float32))
    sh_ref[...] = lax.dot_general(h.astype(jnp.bfloat16),
                                  ws2_ref[...].astype(jnp.bfloat16), _NT_DIMS,
                                  preferred_element_type=jnp.float32)


_shared = pl.pallas_call(
    _shared_kernel,
    grid=(N // TB,),
    in_specs=[
        pl.BlockSpec((TB, D), lambda i: (i, 0)),
        pl.BlockSpec((FS, D), lambda i: (0, 0)),
        pl.BlockSpec((D, FS), lambda i: (0, 0)),
    ],
    out_specs=pl.BlockSpec((TB, D), lambda i: (i, 0)),
    out_shape=jax.ShapeDtypeStruct((N, D), jnp.float32),
)


def _combine_kernel(sh_ref, yp_ref, w_ref, o_ref):
    o_ref[...] = (sh_ref[...] + yp_ref[0] * w_ref[:, 0:1]
                  + yp_ref[1] * w_ref[:, 1:2])


_combine = pl.pallas_call(
    _combine_kernel,
    grid=(N // TB,),
    in_specs=[
        pl.BlockSpec((TB, D), lambda i: (i, 0)),
        pl.BlockSpec((K, TB, D), lambda i: (0, i, 0)),
        pl.BlockSpec((TB, K), lambda i: (i, 0)),
    ],
    out_specs=pl.BlockSpec((TB, D), lambda i: (i, 0)),
    out_shape=jax.ShapeDtypeStruct((N, D), jnp.float32),
)


def kernel(x, W1, W2, Ws1, Ws2, Wg):
    pos, w_pair, tile_e, tile_v = _route(x, Wg)
    pos_flat = pos.reshape(K * N)
    xg = _sc_scatter_x(x, pos_flat)
    sh = _shared(x, Ws1, Ws2)
    y_sorted = _experts(tile_e.reshape(NT), tile_v.reshape(NT), xg, W1, W2)
    ypair = _sc_gather_rows(y_sorted, pos_flat)
    return _combine(sh, ypair.reshape(K, N, D), w_pair)


# speedup vs baseline: 1.4390x; 1.4390x over previous
import functools

import jax
import jax.numpy as jnp
from jax import lax
from jax.experimental import pallas as pl
from jax.experimental.pallas import tpu as pltpu
from jax.experimental.pallas import tpu_sc as plsc

N, D, E, F, S, K = 2048, 1024, 16, 512, 2, 2
FS = F * S
T = 128
NT = (N * K) // T + E
P = NT * T
TB = 256
NC, NS = 2, 16
NW = NC * NS
CH = 64

_NT_DIMS = (((1,), (1,)), ((), ()))


def _silu(v):
    return v * (1.0 / (1.0 + jnp.exp(-v)))


def _route_kernel(x_ref, wg_ref, pos_ref, w_ref, tile_e_ref, tile_v_ref):
    x = x_ref[...]
    wg = wg_ref[...]
    logits = lax.dot_general(x, wg, _NT_DIMS,
                             preferred_element_type=jnp.float32)
    m = jnp.max(logits, axis=1, keepdims=True)
    ex = jnp.exp(logits - m)
    probs = ex / jnp.sum(ex, axis=1, keepdims=True)

    eidx = lax.broadcasted_iota(jnp.int32, (N, E), 1)
    p1 = jnp.max(probs, axis=1, keepdims=True)
    e1 = jnp.min(jnp.where(probs == p1, eidx, E), axis=1, keepdims=True)
    oh1 = eidx == e1
    probs_m = jnp.where(oh1, -1.0, probs)
    p2 = jnp.max(probs_m, axis=1, keepdims=True)
    e2 = jnp.min(jnp.where(probs_m == p2, eidx, E), axis=1, keepdims=True)
    oh2 = eidx == e2
    wsum = p1 + p2
    w_ref[...] = jnp.concatenate([p1 / wsum, p2 / wsum], axis=1)

    c = (oh1 | oh2).astype(jnp.bfloat16)
    cn = 256
    tril = (lax.broadcasted_iota(jnp.int32, (cn, cn), 0)
            >= lax.broadcasted_iota(jnp.int32, (cn, cn), 1)).astype(jnp.bfloat16)
    parts = []
    acc = jnp.zeros((1, E), jnp.float32)
    for ci in range(N // cn):
        sc_ = lax.dot_general(tril, c[ci * cn:(ci + 1) * cn, :],
                              (((1,), (0,)), ((), ())),
                              preferred_element_type=jnp.float32) + acc
        parts.append(sc_)
        acc = sc_[cn - 1:cn, :]
    s_inc = jnp.concatenate(parts, axis=0)
    counts = acc
    tiles_pe = jnp.ceil(counts * (1.0 / T))
    excl = (lax.broadcasted_iota(jnp.int32, (E, E), 0)
            < lax.broadcasted_iota(jnp.int32, (E, E), 1)).astype(jnp.float32)
    tile_start = lax.dot_general(tiles_pe, excl, (((1,), (0,)), ((), ())),
                                 preferred_element_type=jnp.float32)
    total_tiles = jnp.sum(tiles_pe, axis=1, keepdims=True)
    slot_off = tile_start * float(T)

    rank1 = jnp.sum(jnp.where(oh1, s_inc, 0.0), axis=1, keepdims=True) - 1.0
    rank2 = jnp.sum(jnp.where(oh2, s_inc, 0.0), axis=1, keepdims=True) - 1.0
    off_b = jnp.broadcast_to(slot_off, (N, E))
    off1 = jnp.sum(jnp.where(oh1, off_b, 0.0), axis=1, keepdims=True)
    off2 = jnp.sum(jnp.where(oh2, off_b, 0.0), axis=1, keepdims=True)
    pos = jnp.concatenate([jnp.transpose(off1 + rank1),
                           jnp.transpose(off2 + rank2)], axis=0)
    pos_ref[...] = pos.astype(jnp.int32)

    ends = jnp.transpose(tile_start + tiles_pe)
    gi = lax.broadcasted_iota(jnp.int32, (E, NT), 1).astype(jnp.float32)
    g2 = jnp.minimum(gi, total_tiles[0, 0] - 1.0)
    tile_e = jnp.sum((ends <= g2).astype(jnp.float32), axis=0, keepdims=True)
    tile_e_ref[...] = tile_e.astype(jnp.int32)
    tile_v = (lax.broadcasted_iota(jnp.int32, (1, NT), 1).astype(jnp.float32)
              < total_tiles[0, 0])
    tile_v_ref[...] = tile_v.astype(jnp.int32)


_route = pl.pallas_call(
    _route_kernel,
    out_shape=[
        jax.ShapeDtypeStruct((K, N), jnp.int32),
        jax.ShapeDtypeStruct((N, K), jnp.float32),
        jax.ShapeDtypeStruct((1, NT), jnp.int32),
        jax.ShapeDtypeStruct((1, NT), jnp.int32),
    ],
)


def _sc_scatter_x(x, pos_flat):
    n_per_w = N // NW
    nch = n_per_w // CH
    mesh = plsc.VectorSubcoreMesh(core_axis_name="c", subcore_axis_name="s")

    @functools.partial(
        pl.kernel,
        mesh=mesh,
        out_type=jax.ShapeDtypeStruct((P, D), jnp.float32),
        scratch_types=[
            pltpu.VMEM((CH,), jnp.int32),
            pltpu.VMEM((CH,), jnp.int32),
            pltpu.VMEM((CH, D), jnp.float32),
            pltpu.SemaphoreType.DMA,
        ],
    )
    def scatter(x_hbm, pos_hbm, out_hbm, i0_v, i1_v, rows_v, sem):
        wid = lax.axis_index("s") * NC + lax.axis_index("c")
        base = wid * n_per_w
        for ci in range(nch):
            off = base + ci * CH
            pltpu.sync_copy(pos_hbm.at[pl.ds(off, CH)], i0_v)
            pltpu.sync_copy(pos_hbm.at[pl.ds(N + off, CH)], i1_v)
            pltpu.sync_copy(x_hbm.at[pl.ds(off, CH)], rows_v)
            pltpu.async_copy(rows_v, out_hbm.at[i0_v], sem).wait()
            pltpu.async_copy(rows_v, out_hbm.at[i1_v], sem).wait()

    return scatter(x, pos_flat)


def _sc_gather_rows(table, idx):
    b = idx.shape[0]
    b_per_w = b // NW
    ch = min(b_per_w, 64)
    nch = b_per_w // ch
    mesh = plsc.VectorSubcoreMesh(core_axis_name="c", subcore_axis_name="s")

    @functools.partial(
        pl.kernel,
        mesh=mesh,
        out_type=jax.ShapeDtypeStruct((b, D), jnp.float32),
        scratch_types=[
            pltpu.VMEM((ch,), jnp.int32),
            pltpu.VMEM((ch, D), jnp.float32),
            pltpu.SemaphoreType.DMA,
        ],
    )
    def gather(table_hbm, idx_hbm, out_hbm, idx_v, rows_v, sem):
        wid = lax.axis_index("s") * NC + lax.axis_index("c")
        base = wid * b_per_w
        for ci in range(nch):
            off = base + ci * ch
            pltpu.sync_copy(idx_hbm.at[pl.ds(off, ch)], idx_v)
            pltpu.async_copy(table_hbm.at[idx_v], rows_v, sem).wait()
            pltpu.sync_copy(rows_v, out_hbm.at[pl.ds(off, ch)])

    return gather(table, idx)


def _expert_kernel(tile_e_s, tile_v_s, xg_ref, w1_ref, w2_ref, y_ref):
    g = pl.program_id(0)

    @pl.when(tile_v_s[g] != 0)
    def _():
        xgb = xg_ref[...].astype(jnp.bfloat16)
        h = _silu(lax.dot_general(xgb, w1_ref[0].astype(jnp.bfloat16),
                                  _NT_DIMS, preferred_element_type=jnp.float32))
        y = lax.dot_general(h.astype(jnp.bfloat16),
                            w2_ref[0].astype(jnp.bfloat16), _NT_DIMS,
                            preferred_element_type=jnp.float32)
        y_ref[...] = y


_experts = pl.pallas_call(
    _expert_kernel,
    grid_spec=pltpu.PrefetchScalarGridSpec(
        num_scalar_prefetch=2,
        grid=(NT,),
        in_specs=[
            pl.BlockSpec((T, D), lambda g, te, tv: (g, 0)),
            pl.BlockSpec((1, F, D), lambda g, te, tv: (te[g], 0, 0)),
            pl.BlockSpec((1, D, F), lambda g, te, tv: (te[g], 0, 0)),
        ],
        out_specs=pl.BlockSpec((T, D), lambda g, te, tv: (g, 0)),
    ),
    out_shape=jax.ShapeDtypeStruct((P, D), jnp.float32),
)


def _shared_kernel(x_ref, ws1_ref, ws2_ref, sh_ref):
    xb = x_ref[...].astype(jnp.bfloat16)
    h = _silu(lax.dot_general(xb, ws1_ref[...].astype(jnp.bfloat16), _NT_DIMS,
                              preferred_element_type=jnp.float32))
    sh_ref[...] = lax.dot_general(h.astype(jnp.bfloat16),
                                  ws2_ref[...].astype(jnp.bfloat16), _NT_DIMS,
                                  preferred_element_type=jnp.float32)


_shared = pl.pallas_call(
    _shared_kernel,
    grid=(N // TB,),
    in_specs=[
        pl.BlockSpec((TB, D), lambda i: (i, 0)),
        pl.BlockSpec((FS, D), lambda i: (0, 0)),
        pl.BlockSpec((D, FS), lambda i: (0, 0)),
    ],
    out_specs=pl.BlockSpec((TB, D), lambda i: (i, 0)),
    out_shape=jax.ShapeDtypeStruct((N, D), jnp.float32),
)


def _combine_kernel(sh_ref, yp_ref, w_ref, o_ref):
    o_ref[...] = (sh_ref[...] + yp_ref[0] * w_ref[:, 0:1]
                  + yp_ref[1] * w_ref[:, 1:2])


_combine = pl.pallas_call(
    _combine_kernel,
    grid=(N // TB,),
    in_specs=[
        pl.BlockSpec((TB, D), lambda i: (i, 0)),
        pl.BlockSpec((K, TB, D), lambda i: (0, i, 0)),
        pl.BlockSpec((TB, K), lambda i: (i, 0)),
    ],
    out_specs=pl.BlockSpec((TB, D), lambda i: (i, 0)),
    out_shape=jax.ShapeDtypeStruct((N, D), jnp.float32),
)


def kernel(x, W1, W2, Ws1, Ws2, Wg):
    pos, w_pair, tile_e, tile_v = _route(x, Wg)
    pos_flat = pos.reshape(K * N)
    xg = _sc_scatter_x(x, pos_flat)
    sh = _shared(x, Ws1, Ws2)
    y_sorted = _experts(tile_e.reshape(NT), tile_v.reshape(NT), xg, W1, W2)
    ypair = _sc_gather_rows(y_sorted, pos_flat)
    return _combine(sh, ypair.reshape(K, N, D), w_pair)


# speedup vs baseline: 1.6435x; 1.1421x over previous
import functools

import jax
import jax.numpy as jnp
from jax import lax
from jax.experimental import pallas as pl
from jax.experimental.pallas import tpu as pltpu
from jax.experimental.pallas import tpu_sc as plsc

N, D, E, F, S, K = 2048, 1024, 16, 512, 2, 2
FS = F * S
T = 256
NT = (N * K) // T + E
P = NT * T
TB = 256
NC, NS = 2, 16
NW = NC * NS
CH = 64

_NT_DIMS = (((1,), (1,)), ((), ()))


def _silu(v):
    return v * (1.0 / (1.0 + jnp.exp(-v)))


def _route_kernel(x_ref, wg_ref, pos_ref, w_ref, tile_e_ref, tile_v_ref):
    x = x_ref[...]
    wg = wg_ref[...]
    logits = lax.dot_general(x, wg, _NT_DIMS,
                             preferred_element_type=jnp.float32)
    m = jnp.max(logits, axis=1, keepdims=True)
    ex = jnp.exp(logits - m)
    probs = ex / jnp.sum(ex, axis=1, keepdims=True)

    eidx = lax.broadcasted_iota(jnp.int32, (N, E), 1)
    p1 = jnp.max(probs, axis=1, keepdims=True)
    e1 = jnp.min(jnp.where(probs == p1, eidx, E), axis=1, keepdims=True)
    oh1 = eidx == e1
    probs_m = jnp.where(oh1, -1.0, probs)
    p2 = jnp.max(probs_m, axis=1, keepdims=True)
    e2 = jnp.min(jnp.where(probs_m == p2, eidx, E), axis=1, keepdims=True)
    oh2 = eidx == e2
    wsum = p1 + p2
    w_ref[...] = jnp.concatenate([p1 / wsum, p2 / wsum], axis=1)

    c = (oh1 | oh2).astype(jnp.bfloat16)
    cn = 256
    tril = (lax.broadcasted_iota(jnp.int32, (cn, cn), 0)
            >= lax.broadcasted_iota(jnp.int32, (cn, cn), 1)).astype(jnp.bfloat16)
    parts = []
    acc = jnp.zeros((1, E), jnp.float32)
    for ci in range(N // cn):
        sc_ = lax.dot_general(tril, c[ci * cn:(ci + 1) * cn, :],
                              (((1,), (0,)), ((), ())),
                              preferred_element_type=jnp.float32) + acc
        parts.append(sc_)
        acc = sc_[cn - 1:cn, :]
    s_inc = jnp.concatenate(parts, axis=0)
    counts = acc
    tiles_pe = jnp.ceil(counts * (1.0 / T))
    excl = (lax.broadcasted_iota(jnp.int32, (E, E), 0)
            < lax.broadcasted_iota(jnp.int32, (E, E), 1)).astype(jnp.float32)
    tile_start = lax.dot_general(tiles_pe, excl, (((1,), (0,)), ((), ())),
                                 preferred_element_type=jnp.float32)
    total_tiles = jnp.sum(tiles_pe, axis=1, keepdims=True)
    slot_off = tile_start * float(T)

    rank1 = jnp.sum(jnp.where(oh1, s_inc, 0.0), axis=1, keepdims=True) - 1.0
    rank2 = jnp.sum(jnp.where(oh2, s_inc, 0.0), axis=1, keepdims=True) - 1.0
    off_b = jnp.broadcast_to(slot_off, (N, E))
    off1 = jnp.sum(jnp.where(oh1, off_b, 0.0), axis=1, keepdims=True)
    off2 = jnp.sum(jnp.where(oh2, off_b, 0.0), axis=1, keepdims=True)
    pos = jnp.concatenate([jnp.transpose(off1 + rank1),
                           jnp.transpose(off2 + rank2)], axis=0)
    pos_ref[...] = pos.astype(jnp.int32)

    ends = jnp.transpose(tile_start + tiles_pe)
    gi = lax.broadcasted_iota(jnp.int32, (E, NT), 1).astype(jnp.float32)
    g2 = jnp.minimum(gi, total_tiles[0, 0] - 1.0)
    tile_e = jnp.sum((ends <= g2).astype(jnp.float32), axis=0, keepdims=True)
    tile_e_ref[...] = tile_e.astype(jnp.int32)
    tile_v = (lax.broadcasted_iota(jnp.int32, (1, NT), 1).astype(jnp.float32)
              < total_tiles[0, 0])
    tile_v_ref[...] = tile_v.astype(jnp.int32)


_route = pl.pallas_call(
    _route_kernel,
    out_shape=[
        jax.ShapeDtypeStruct((K, N), jnp.int32),
        jax.ShapeDtypeStruct((N, K), jnp.float32),
        jax.ShapeDtypeStruct((1, NT), jnp.int32),
        jax.ShapeDtypeStruct((1, NT), jnp.int32),
    ],
)


def _sc_scatter_x(x, pos_flat):
    n_per_w = N // NW
    nch = n_per_w // CH
    mesh = plsc.VectorSubcoreMesh(core_axis_name="c", subcore_axis_name="s")

    @functools.partial(
        pl.kernel,
        mesh=mesh,
        out_type=jax.ShapeDtypeStruct((P, D), jnp.float32),
        scratch_types=[
            pltpu.VMEM((CH,), jnp.int32),
            pltpu.VMEM((CH,), jnp.int32),
            pltpu.VMEM((CH, D), jnp.float32),
            pltpu.SemaphoreType.DMA,
        ],
    )
    def scatter(x_hbm, pos_hbm, out_hbm, i0_v, i1_v, rows_v, sem):
        wid = lax.axis_index("s") * NC + lax.axis_index("c")
        base = wid * n_per_w
        for ci in range(nch):
            off = base + ci * CH
            pltpu.sync_copy(pos_hbm.at[pl.ds(off, CH)], i0_v)
            pltpu.sync_copy(pos_hbm.at[pl.ds(N + off, CH)], i1_v)
            pltpu.sync_copy(x_hbm.at[pl.ds(off, CH)], rows_v)
            pltpu.async_copy(rows_v, out_hbm.at[i0_v], sem).wait()
            pltpu.async_copy(rows_v, out_hbm.at[i1_v], sem).wait()

    return scatter(x, pos_flat)


def _sc_gather_rows(table, idx):
    b = idx.shape[0]
    b_per_w = b // NW
    ch = min(b_per_w, 64)
    nch = b_per_w // ch
    mesh = plsc.VectorSubcoreMesh(core_axis_name="c", subcore_axis_name="s")

    @functools.partial(
        pl.kernel,
        mesh=mesh,
        out_type=jax.ShapeDtypeStruct((b, D), jnp.float32),
        scratch_types=[
            pltpu.VMEM((ch,), jnp.int32),
            pltpu.VMEM((ch, D), jnp.float32),
            pltpu.SemaphoreType.DMA,
        ],
    )
    def gather(table_hbm, idx_hbm, out_hbm, idx_v, rows_v, sem):
        wid = lax.axis_index("s") * NC + lax.axis_index("c")
        base = wid * b_per_w
        for ci in range(nch):
            off = base + ci * ch
            pltpu.sync_copy(idx_hbm.at[pl.ds(off, ch)], idx_v)
            pltpu.async_copy(table_hbm.at[idx_v], rows_v, sem).wait()
            pltpu.sync_copy(rows_v, out_hbm.at[pl.ds(off, ch)])

    return gather(table, idx)


def _expert_kernel(tile_e_s, tile_v_s, xg_ref, w1_ref, w2_ref, y_ref):
    g = pl.program_id(0)

    @pl.when(tile_v_s[g] != 0)
    def _():
        xgb = xg_ref[...].astype(jnp.bfloat16)
        h = _silu(lax.dot_general(xgb, w1_ref[0].astype(jnp.bfloat16),
                                  _NT_DIMS, preferred_element_type=jnp.float32))
        y = lax.dot_general(h.astype(jnp.bfloat16),
                            w2_ref[0].astype(jnp.bfloat16), _NT_DIMS,
                            preferred_element_type=jnp.float32)
        y_ref[...] = y


_experts = pl.pallas_call(
    _expert_kernel,
    grid_spec=pltpu.PrefetchScalarGridSpec(
        num_scalar_prefetch=2,
        grid=(NT,),
        in_specs=[
            pl.BlockSpec((T, D), lambda g, te, tv: (g, 0)),
            pl.BlockSpec((1, F, D), lambda g, te, tv: (te[g], 0, 0)),
            pl.BlockSpec((1, D, F), lambda g, te, tv: (te[g], 0, 0)),
        ],
        out_specs=pl.BlockSpec((T, D), lambda g, te, tv: (g, 0)),
    ),
    out_shape=jax.ShapeDtypeStruct((P, D), jnp.float32),
)


def _shared_kernel(x_ref, ws1_ref, ws2_ref, sh_ref):
    xb = x_ref[...].astype(jnp.bfloat16)
    h = _silu(lax.dot_general(xb, ws1_ref[...].astype(jnp.bfloat16), _NT_DIMS,
                              preferred_element_type=jnp.float32))
    sh_ref[...] = lax.dot_general(h.astype(jnp.bfloat16),
                                  ws2_ref[...].astype(jnp.bfloat16), _NT_DIMS,
                                  preferred_element_type=jnp.float32)


_shared = pl.pallas_call(
    _shared_kernel,
    grid=(N // TB,),
    in_specs=[
        pl.BlockSpec((TB, D), lambda i: (i, 0)),
        pl.BlockSpec((FS, D), lambda i: (0, 0)),
        pl.BlockSpec((D, FS), lambda i: (0, 0)),
    ],
    out_specs=pl.BlockSpec((TB, D), lambda i: (i, 0)),
    out_shape=jax.ShapeDtypeStruct((N, D), jnp.float32),
)


def _combine_kernel(sh_ref, yp_ref, w_ref, o_ref):
    o_ref[...] = (sh_ref[...] + yp_ref[0] * w_ref[:, 0:1]
                  + yp_ref[1] * w_ref[:, 1:2])


_combine = pl.pallas_call(
    _combine_kernel,
    grid=(N // TB,),
    in_specs=[
        pl.BlockSpec((TB, D), lambda i: (i, 0)),
        pl.BlockSpec((K, TB, D), lambda i: (0, i, 0)),
        pl.BlockSpec((TB, K), lambda i: (i, 0)),
    ],
    out_specs=pl.BlockSpec((TB, D), lambda i: (i, 0)),
    out_shape=jax.ShapeDtypeStruct((N, D), jnp.float32),
)


def kernel(x, W1, W2, Ws1, Ws2, Wg):
    pos, w_pair, tile_e, tile_v = _route(x, Wg)
    pos_flat = pos.reshape(K * N)
    xg = _sc_scatter_x(x, pos_flat)
    sh = _shared(x, Ws1, Ws2)
    y_sorted = _experts(tile_e.reshape(NT), tile_v.reshape(NT), xg, W1, W2)
    ypair = _sc_gather_rows(y_sorted, pos_flat)
    return _combine(sh, ypair.reshape(K, N, D), w_pair)


# speedup vs baseline: 1.6630x; 1.0119x over previous
import functools

import jax
import jax.numpy as jnp
from jax import lax
from jax.experimental import pallas as pl
from jax.experimental.pallas import tpu as pltpu
from jax.experimental.pallas import tpu_sc as plsc

N, D, E, F, S, K = 2048, 1024, 16, 512, 2, 2
FS = F * S
T = 256
NT = (N * K) // T + E
P = NT * T
TB = 256
NC, NS = 2, 16
NW = NC * NS
CH = 64

_NT_DIMS = (((1,), (1,)), ((), ()))


def _silu(v):
    return v * (1.0 / (1.0 + jnp.exp(-v)))


def _route_kernel(x_ref, wg_ref, pos_ref, w_ref, tile_e_ref, tile_v_ref):
    x = x_ref[...]
    wg = wg_ref[...]
    logits = lax.dot_general(x, wg, _NT_DIMS,
                             preferred_element_type=jnp.float32)
    m = jnp.max(logits, axis=1, keepdims=True)
    ex = jnp.exp(logits - m)
    probs = ex / jnp.sum(ex, axis=1, keepdims=True)

    eidx = lax.broadcasted_iota(jnp.int32, (N, E), 1)
    p1 = jnp.max(probs, axis=1, keepdims=True)
    e1 = jnp.min(jnp.where(probs == p1, eidx, E), axis=1, keepdims=True)
    oh1 = eidx == e1
    probs_m = jnp.where(oh1, -1.0, probs)
    p2 = jnp.max(probs_m, axis=1, keepdims=True)
    e2 = jnp.min(jnp.where(probs_m == p2, eidx, E), axis=1, keepdims=True)
    oh2 = eidx == e2
    wsum = p1 + p2
    w_ref[...] = jnp.concatenate([p1 / wsum, p2 / wsum], axis=1)

    c = (oh1 | oh2).astype(jnp.bfloat16)
    cn = 256
    tril = (lax.broadcasted_iota(jnp.int32, (cn, cn), 0)
            >= lax.broadcasted_iota(jnp.int32, (cn, cn), 1)).astype(jnp.bfloat16)
    parts = []
    acc = jnp.zeros((1, E), jnp.float32)
    for ci in range(N // cn):
        sc_ = lax.dot_general(tril, c[ci * cn:(ci + 1) * cn, :],
                              (((1,), (0,)), ((), ())),
                              preferred_element_type=jnp.float32) + acc
        parts.append(sc_)
        acc = sc_[cn - 1:cn, :]
    s_inc = jnp.concatenate(parts, axis=0)
    counts = acc
    tiles_pe = jnp.ceil(counts * (1.0 / T))
    excl = (lax.broadcasted_iota(jnp.int32, (E, E), 0)
            < lax.broadcasted_iota(jnp.int32, (E, E), 1)).astype(jnp.float32)
    tile_start = lax.dot_general(tiles_pe, excl, (((1,), (0,)), ((), ())),
                                 preferred_element_type=jnp.float32)
    total_tiles = jnp.sum(tiles_pe, axis=1, keepdims=True)
    slot_off = tile_start * float(T)

    rank1 = jnp.sum(jnp.where(oh1, s_inc, 0.0), axis=1, keepdims=True) - 1.0
    rank2 = jnp.sum(jnp.where(oh2, s_inc, 0.0), axis=1, keepdims=True) - 1.0
    off_b = jnp.broadcast_to(slot_off, (N, E))
    off1 = jnp.sum(jnp.where(oh1, off_b, 0.0), axis=1, keepdims=True)
    off2 = jnp.sum(jnp.where(oh2, off_b, 0.0), axis=1, keepdims=True)
    pos = jnp.concatenate([jnp.transpose(off1 + rank1),
                           jnp.transpose(off2 + rank2)], axis=1)
    pos_ref[...] = pos.astype(jnp.int32)

    ends = jnp.transpose(tile_start + tiles_pe)
    gi = lax.broadcasted_iota(jnp.int32, (E, NT), 1).astype(jnp.float32)
    g2 = jnp.minimum(gi, total_tiles[0, 0] - 1.0)
    tile_e = jnp.sum((ends <= g2).astype(jnp.float32), axis=0, keepdims=True)
    tile_e_ref[...] = tile_e.astype(jnp.int32)
    tile_v = (lax.broadcasted_iota(jnp.int32, (1, NT), 1).astype(jnp.float32)
              < total_tiles[0, 0])
    tile_v_ref[...] = tile_v.astype(jnp.int32)


_route = pl.pallas_call(
    _route_kernel,
    out_shape=[
        jax.ShapeDtypeStruct((1, K * N), jnp.int32),
        jax.ShapeDtypeStruct((N, K), jnp.float32),
        jax.ShapeDtypeStruct((1, NT), jnp.int32),
        jax.ShapeDtypeStruct((1, NT), jnp.int32),
    ],
)


def _sc_scatter_x(x, pos_flat):
    n_per_w = N // NW
    nch = n_per_w // CH
    mesh = plsc.VectorSubcoreMesh(core_axis_name="c", subcore_axis_name="s")

    @functools.partial(
        pl.kernel,
        mesh=mesh,
        out_type=jax.ShapeDtypeStruct((P, D), jnp.float32),
        scratch_types=[
            pltpu.VMEM((CH,), jnp.int32),
            pltpu.VMEM((CH,), jnp.int32),
            pltpu.VMEM((CH, D), jnp.float32),
            pltpu.SemaphoreType.DMA,
        ],
    )
    def scatter(x_hbm, pos_hbm, out_hbm, i0_v, i1_v, rows_v, sem):
        wid = lax.axis_index("s") * NC + lax.axis_index("c")
        base = wid * n_per_w
        for ci in range(nch):
            off = base + ci * CH
            pltpu.sync_copy(pos_hbm.at[0, pl.ds(off, CH)], i0_v)
            pltpu.sync_copy(pos_hbm.at[0, pl.ds(N + off, CH)], i1_v)
            pltpu.sync_copy(x_hbm.at[pl.ds(off, CH)], rows_v)
            pltpu.async_copy(rows_v, out_hbm.at[i0_v], sem).wait()
            pltpu.async_copy(rows_v, out_hbm.at[i1_v], sem).wait()

    return scatter(x, pos_flat)


def _sc_gather_rows(table, idx):
    b = idx.shape[1]
    b_per_w = b // NW
    ch = min(b_per_w, 64)
    nch = b_per_w // ch
    mesh = plsc.VectorSubcoreMesh(core_axis_name="c", subcore_axis_name="s")

    @functools.partial(
        pl.kernel,
        mesh=mesh,
        out_type=jax.ShapeDtypeStruct((b, D), jnp.float32),
        scratch_types=[
            pltpu.VMEM((ch,), jnp.int32),
            pltpu.VMEM((ch, D), jnp.float32),
            pltpu.SemaphoreType.DMA,
        ],
    )
    def gather(table_hbm, idx_hbm, out_hbm, idx_v, rows_v, sem):
        wid = lax.axis_index("s") * NC + lax.axis_index("c")
        base = wid * b_per_w
        for ci in range(nch):
            off = base + ci * ch
            pltpu.sync_copy(idx_hbm.at[0, pl.ds(off, ch)], idx_v)
            pltpu.async_copy(table_hbm.at[idx_v], rows_v, sem).wait()
            pltpu.sync_copy(rows_v, out_hbm.at[pl.ds(off, ch)])

    return gather(table, idx)


def _expert_kernel(tile_e_s, tile_v_s, xg_ref, w1_ref, w2_ref, y_ref):
    g = pl.program_id(0)

    @pl.when(tile_v_s[0, g] != 0)
    def _():
        xgb = xg_ref[...].astype(jnp.bfloat16)
        h = _silu(lax.dot_general(xgb, w1_ref[0].astype(jnp.bfloat16),
                                  _NT_DIMS, preferred_element_type=jnp.float32))
        y = lax.dot_general(h.astype(jnp.bfloat16),
                            w2_ref[0].astype(jnp.bfloat16), _NT_DIMS,
                            preferred_element_type=jnp.float32)
        y_ref[...] = y


_experts = pl.pallas_call(
    _expert_kernel,
    grid_spec=pltpu.PrefetchScalarGridSpec(
        num_scalar_prefetch=2,
        grid=(NT,),
        in_specs=[
            pl.BlockSpec((T, D), lambda g, te, tv: (g, 0)),
            pl.BlockSpec((1, F, D), lambda g, te, tv: (te[0, g], 0, 0)),
            pl.BlockSpec((1, D, F), lambda g, te, tv: (te[0, g], 0, 0)),
        ],
        out_specs=pl.BlockSpec((T, D), lambda g, te, tv: (g, 0)),
    ),
    out_shape=jax.ShapeDtypeStruct((P, D), jnp.float32),
)


def _shared_kernel(x_ref, ws1_ref, ws2_ref, sh_ref):
    xb = x_ref[...].astype(jnp.bfloat16)
    h = _silu(lax.dot_general(xb, ws1_ref[...].astype(jnp.bfloat16), _NT_DIMS,
                              preferred_element_type=jnp.float32))
    sh_ref[...] = lax.dot_general(h.astype(jnp.bfloat16),
                                  ws2_ref[...].astype(jnp.bfloat16), _NT_DIMS,
                                  preferred_element_type=jnp.float32)


_shared = pl.pallas_call(
    _shared_kernel,
    grid=(N // TB,),
    in_specs=[
        pl.BlockSpec((TB, D), lambda i: (i, 0)),
        pl.BlockSpec((FS, D), lambda i: (0, 0)),
        pl.BlockSpec((D, FS), lambda i: (0, 0)),
    ],
    out_specs=pl.BlockSpec((TB, D), lambda i: (i, 0)),
    out_shape=jax.ShapeDtypeStruct((N, D), jnp.float32),
)


def _combine_kernel(sh_ref, yp_ref, w_ref, o_ref):
    o_ref[...] = (sh_ref[...] + yp_ref[0] * w_ref[:, 0:1]
                  + yp_ref[1] * w_ref[:, 1:2])


_combine = pl.pallas_call(
    _combine_kernel,
    grid=(N // TB,),
    in_specs=[
        pl.BlockSpec((TB, D), lambda i: (i, 0)),
        pl.BlockSpec((K, TB, D), lambda i: (0, i, 0)),
        pl.BlockSpec((TB, K), lambda i: (i, 0)),
    ],
    out_specs=pl.BlockSpec((TB, D), lambda i: (i, 0)),
    out_shape=jax.ShapeDtypeStruct((N, D), jnp.float32),
)


def kernel(x, W1, W2, Ws1, Ws2, Wg):
    pos, w_pair, tile_e, tile_v = _route(x, Wg)
    xg = _sc_scatter_x(x, pos)
    sh = _shared(x, Ws1, Ws2)
    y_sorted = _experts(tile_e, tile_v, xg, W1, W2)
    ypair = _sc_gather_rows(y_sorted, pos)
    return _combine(sh, ypair.reshape(K, N, D), w_pair)
